# baseline (speedup 1.0000x reference)
import functools
import jax
import jax.numpy as jnp
from jax import lax
from jax.experimental import pallas as pl
from jax.experimental.pallas import tpu as pltpu

N = 4096
R = 128
NNZ = 131072
BLK = 256
GRID = N // BLK
F32 = jnp.float32
BIGI = 2 ** 30


def _phase_a(adj_ref, isf_ref, usvf_ref, lp_ref, uf_ref, cmin_ref, csum_ref):
    i = pl.program_id(0)
    uf = jnp.dot(adj_ref[...], isf_ref[...], preferred_element_type=F32)
    uf = uf * (1.0 / lp_ref[...])
    uf_ref[...] = uf
    e = lax.dot_general(uf, usvf_ref[...], (((1,), (1,)), ((), ())),
                        preferred_element_type=F32)
    bmin = jnp.min(e, axis=0, keepdims=True)
    bsum = jnp.sum(e, axis=0, keepdims=True)

    @pl.when(i == 0)
    def _():
        cmin_ref[...] = bmin
        csum_ref[...] = bsum

    @pl.when(i != 0)
    def _():
        cmin_ref[...] = jnp.minimum(cmin_ref[...], bmin)
        csum_ref[...] = csum_ref[...] + bsum


def _run_a(adj_mat, item_sv_f, user_sv_f, lambda_mat_pos):
    return pl.pallas_call(
        _phase_a,
        grid=(GRID,),
        in_specs=[
            pl.BlockSpec((BLK, N), lambda i: (i, 0)),
            pl.BlockSpec((N, R), lambda i: (0, 0)),
            pl.BlockSpec((N, R), lambda i: (0, 0)),
            pl.BlockSpec((1, R), lambda i: (0, 0)),
        ],
        out_specs=[
            pl.BlockSpec((BLK, R), lambda i: (i, 0)),
            pl.BlockSpec((1, N), lambda i: (0, 0)),
            pl.BlockSpec((1, N), lambda i: (0, 0)),
        ],
        out_shape=[
            jax.ShapeDtypeStruct((N, R), F32),
            jax.ShapeDtypeStruct((1, N), F32),
            jax.ShapeDtypeStruct((1, N), F32),
        ],
    )(adj_mat, item_sv_f, user_sv_f, lambda_mat_pos.reshape(1, R))


def _phase_b(igp_ref, ugp_ref, usv_ref, isv_ref, lam_ref, fp0_ref, fp1_ref,
             gu_ref, gis_ref, fp_ref):
    inv_coeff = 1.0 / 4096.0
    ig = (igp_ref[0] + igp_ref[1]) * inv_coeff
    ug = (ugp_ref[0] + ugp_ref[1]) * inv_coeff
    gu_ref[...] = (ug + usv_ref[...]) * 0.5
    gis_ref[...] = ((ig + isv_ref[...]) * 0.5) * (1.0 / lam_ref[...])
    fp_ref[...] = jnp.minimum(fp0_ref[...], fp1_ref[...])


def _run_b(ig_parts, ug_parts, user_sv, item_sv, lambda_mat, fp0, fp1):
    return pl.pallas_call(
        _phase_b,
        out_shape=[
            jax.ShapeDtypeStruct((N, R), F32),
            jax.ShapeDtypeStruct((N, R), F32),
            jax.ShapeDtypeStruct((N, 1), jnp.int32),
        ],
    )(ig_parts, ug_parts, user_sv, item_sv, lambda_mat.reshape(1, R),
      fp0.reshape(N, 1), fp1.reshape(N, 1))


def _phase_c(na_ref, uf_ref, gis_ref, usvf_ref, hd0_ref, hd1_ref,
             cmin_ref, csum_ref, c1_ref, v1_ref, fp_ref, out_ref):
    w = jnp.dot(na_ref[...], gis_ref[...], preferred_element_type=F32)
    hd = hd0_ref[...] + hd1_ref[...]
    rating = lax.dot_general(w, hd, (((1,), (1,)), ((), ())),
                             preferred_element_type=F32)
    e = lax.dot_general(uf_ref[...], usvf_ref[...], (((1,), (1,)), ((), ())),
                        preferred_element_type=F32)
    cmin = cmin_ref[...]
    inv = 1.0 / (csum_ref[...] - float(N) * cmin + float(N) * 1e-8)
    ne = (e - cmin + 1e-8) * inv
    c1 = c1_ref[...]
    c2 = lax.rem(c1 + 1, N)
    v1 = v1_ref[...]
    has = fp_ref[...] < jnp.int32(NNZ)
    d1 = jnp.where(v1 == 0.0, 1e-5, v1)
    d2 = jnp.where(v1 == 0.0, 1e-5, -v1)
    s1 = jnp.where(has, 0.2 / d1, 0.0)
    s2 = jnp.where(has, 0.2 / d2, 0.0)
    iota = lax.broadcasted_iota(jnp.int32, (BLK, N), 1)
    adj = ne * (jnp.where(iota == c1, s1, 0.0) + jnp.where(iota == c2, s2, 0.0))
    out_ref[...] = rating + adj


def _run_c(norm_adj, uf, gis, user_sv_f, hd0, hd1, cmin, csum, c1, v1, fp):
    return pl.pallas_call(
        _phase_c,
        grid=(GRID,),
        in_specs=[
            pl.BlockSpec((BLK, N), lambda i: (i, 0)),
            pl.BlockSpec((BLK, R), lambda i: (i, 0)),
            pl.BlockSpec((N, R), lambda i: (0, 0)),
            pl.BlockSpec((N, R), lambda i: (0, 0)),
            pl.BlockSpec((N, R), lambda i: (0, 0)),
            pl.BlockSpec((N, R), lambda i: (0, 0)),
            pl.BlockSpec((1, N), lambda i: (0, 0)),
            pl.BlockSpec((1, N), lambda i: (0, 0)),
            pl.BlockSpec((BLK, 1), lambda i: (i, 0)),
            pl.BlockSpec((BLK, 1), lambda i: (i, 0)),
            pl.BlockSpec((BLK, 1), lambda i: (i, 0)),
        ],
        out_specs=pl.BlockSpec((BLK, N), lambda i: (i, 0)),
        out_shape=jax.ShapeDtypeStruct((N, N), F32),
    )(norm_adj, uf, gis, user_sv_f, hd0, hd1, cmin, csum,
      c1.reshape(N, 1), v1.reshape(N, 1), fp.reshape(N, 1))


def kernel(adj_mat, norm_adj, user_sv, item_sv, user_sv_f, item_sv_f,
           pos_values, lambda_mat, lambda_mat_pos, pos_indices):
    rows = pos_indices[0]
    cols = pos_indices[1]

    uf, cmin, csum = _run_a(adj_mat, item_sv_f, user_sv_f, lambda_mat_pos)

    ig_raw = jax.ops.segment_sum(pos_values[:, None] * user_sv[rows], cols,
                                 num_segments=N)
    ug_raw = jax.ops.segment_sum(pos_values[:, None] * item_sv[cols], rows,
                                 num_segments=N)
    order = jnp.arange(NNZ, dtype=jnp.int32)
    fp = jax.ops.segment_min(order, rows, num_segments=N)
    fp = jnp.minimum(fp, jnp.int32(BIGI))
    zeros_p = jnp.zeros((N, R), F32)

    gu, gis, fpm = _run_b(jnp.stack([ig_raw, zeros_p]),
                          jnp.stack([ug_raw, zeros_p]),
                          user_sv, item_sv, lambda_mat, fp,
                          jnp.full((N,), BIGI, jnp.int32))
    fpm = fpm.reshape(N)
    fpc = jnp.minimum(fpm, NNZ - 1)
    c1 = cols[fpc]
    v1 = pos_values[fpc]
    hd_raw = jax.ops.segment_sum(pos_values[:, None] * gu[rows], cols,
                                 num_segments=N)

    out = _run_c(norm_adj, uf, gis, user_sv_f, hd_raw, zeros_p,
                 cmin, csum, c1, v1, fpm)
    return out


# speedup vs baseline: 6.8903x; 6.8903x over previous
import functools
import jax
import jax.numpy as jnp
from jax import lax
from jax.experimental import pallas as pl
from jax.experimental.pallas import tpu as pltpu
from jax.experimental.pallas import tpu_sc as plsc

N = 4096
R = 128
NNZ = 131072
BLK = 256
GRID = N // BLK
F32 = jnp.float32
BIGI = 2 ** 30

NW = 32
EW = NNZ // NW
KS = 128
NSUB = 2
SUP = KS * NSUB
NCH = EW // SUP
RPW = N // 16


def _phase_a(adj_ref, isf_ref, usvf_ref, lp_ref, uf_ref, cmin_ref, csum_ref):
    i = pl.program_id(0)
    uf = jnp.dot(adj_ref[...], isf_ref[...], preferred_element_type=F32)
    uf = uf * (1.0 / lp_ref[...])
    uf_ref[...] = uf
    e = lax.dot_general(uf, usvf_ref[...], (((1,), (1,)), ((), ())),
                        preferred_element_type=F32)
    bmin = jnp.min(e, axis=0, keepdims=True)
    bsum = jnp.sum(e, axis=0, keepdims=True)

    @pl.when(i == 0)
    def _():
        cmin_ref[...] = bmin
        csum_ref[...] = bsum

    @pl.when(i != 0)
    def _():
        cmin_ref[...] = jnp.minimum(cmin_ref[...], bmin)
        csum_ref[...] = csum_ref[...] + bsum


def _run_a(adj_mat, item_sv_f, user_sv_f, lambda_mat_pos):
    return pl.pallas_call(
        _phase_a,
        grid=(GRID,),
        in_specs=[
            pl.BlockSpec((BLK, N), lambda i: (i, 0)),
            pl.BlockSpec((N, R), lambda i: (0, 0)),
            pl.BlockSpec((N, R), lambda i: (0, 0)),
            pl.BlockSpec((1, R), lambda i: (0, 0)),
        ],
        out_specs=[
            pl.BlockSpec((BLK, R), lambda i: (i, 0)),
            pl.BlockSpec((1, N), lambda i: (0, 0)),
            pl.BlockSpec((1, N), lambda i: (0, 0)),
        ],
        out_shape=[
            jax.ShapeDtypeStruct((N, R), F32),
            jax.ShapeDtypeStruct((1, N), F32),
            jax.ShapeDtypeStruct((1, N), F32),
        ],
    )(adj_mat, item_sv_f, user_sv_f, lambda_mat_pos.reshape(1, R))


def _zero_vmem_block(zbuf, nrow):
    def zrow(i, _):
        for j in range(R // 16):
            zbuf[i, pl.ds(j * 16, 16)] = jnp.zeros((16,), F32)
        return 0
    lax.fori_loop(0, nrow, zrow, 0)


def _scale_rows(buf, vbuf, j):
    def body(g, _):
        v16 = vbuf[j, pl.ds(g * 16, 16)]
        for l in range(16):
            k = g * 16 + l
            v = v16[l]
            for t in range(R // 16):
                sl = pl.ds(t * 16, 16)
                buf[j, k, sl] = buf[j, k, sl] * v
        return 0
    lax.fori_loop(0, KS // 16, body, 0)


def _fp_update(fp_loc, ridx, j, ebase):
    def body(g, _):
        r16 = ridx[j, pl.ds(g * 16, 16)]
        e16 = (ebase + g * 16) + lax.broadcasted_iota(jnp.int32, (16,), 0)
        idx = lax.broadcasted_iota(jnp.int32, (16,), 0)
        for l in range(16):
            m = idx == l
            cur = plsc.load_gather(fp_loc, [r16], mask=m)
            plsc.store_scatter(fp_loc, [r16], jnp.minimum(cur, e16), mask=m)
        return 0
    lax.fori_loop(0, KS // 16, body, 0)


def _sc_edges_body(rows_h, cols_h, vals_h, item_h, user_h,
                   ig_h, ug_h, fp_h,
                   ridx, cidx, vbuf, buf, fp_loc, zbuf,
                   acc_sh, sem_in, sem_g, sem_s):
    c = lax.axis_index("c")
    s = lax.axis_index("s")
    w = c * 16 + s
    wbase = w * EW
    base_r = s * RPW
    sl_out = pl.ds(base_r, RPW)

    _zero_vmem_block(zbuf, KS)

    def fz(i, _):
        fp_loc[pl.ds(i * 16, 16)] = jnp.full((16,), BIGI, jnp.int32)
        return 0
    lax.fori_loop(0, N // 16, fz, 0)

    for phase in range(2):
        for half in range(RPW // KS):
            pltpu.sync_copy(zbuf, acc_sh.at[pl.ds(base_r + half * KS, KS)])
        plsc.subcore_barrier()

        def chunk(ch, _):
            ebase = wbase + ch * SUP
            cps = []
            for j in range(NSUB):
                sl = pl.ds(ebase + j * KS, KS)
                cps.append(pltpu.async_copy(rows_h.at[sl], ridx.at[j], sem_in))
                cps.append(pltpu.async_copy(cols_h.at[sl], cidx.at[j], sem_in))
                cps.append(pltpu.async_copy(vals_h.at[sl], vbuf.at[j], sem_in))
            for cp in cps:
                cp.wait()
            tbl = item_h if phase == 0 else user_h
            gi = cidx if phase == 0 else ridx
            si = ridx if phase == 0 else cidx
            gps = [pltpu.async_copy(tbl.at[gi.at[j]], buf.at[j], sem_g)
                   for j in range(NSUB)]
            for cp in gps:
                cp.wait()
            for j in range(NSUB):
                _scale_rows(buf, vbuf, j)
                if phase == 0:
                    _fp_update(fp_loc, ridx, j, ebase + j * KS)
            sps = [pltpu.async_copy(buf.at[j], acc_sh.at[si.at[j]],
                                    sem_s, add=True)
                   for j in range(NSUB)]
            for cp in sps:
                cp.wait()
            return 0
        lax.fori_loop(0, NCH, chunk, 0)
        plsc.subcore_barrier()
        dst = ug_h if phase == 0 else ig_h
        pltpu.sync_copy(acc_sh.at[sl_out], dst.at[c, sl_out])
        plsc.subcore_barrier()

    pltpu.sync_copy(fp_loc, fp_h.at[w])


def _run_sc_edges(rows, cols, vals, item_sv, user_sv):
    mesh = plsc.VectorSubcoreMesh(core_axis_name="c", subcore_axis_name="s")
    f = pl.kernel(
        _sc_edges_body,
        compiler_params=pltpu.CompilerParams(needs_layout_passes=False),
        out_type=[
            jax.ShapeDtypeStruct((2, N, R), F32),
            jax.ShapeDtypeStruct((2, N, R), F32),
            jax.ShapeDtypeStruct((NW, N), jnp.int32),
        ],
        mesh=mesh,
        scratch_types=[
            pltpu.VMEM((NSUB, KS), jnp.int32),
            pltpu.VMEM((NSUB, KS), jnp.int32),
            pltpu.VMEM((NSUB, KS), F32),
            pltpu.VMEM((NSUB, KS, R), F32),
            pltpu.VMEM((N,), jnp.int32),
            pltpu.VMEM((KS, R), F32),
            pltpu.VMEM_SHARED((N, R), F32),
            pltpu.SemaphoreType.DMA,
            pltpu.SemaphoreType.DMA,
            pltpu.SemaphoreType.DMA,
        ],
    )
    return f(rows, cols, vals, item_sv, user_sv)


def _sc_hd_body(rows_h, cols_h, vals_h, gu_h, fp_h,
                hd_h, c1_h, v1_h,
                ridx, cidx, vbuf, buf_g, iidx, cbuf, vvbuf, zbuf,
                hd_sh, sem_in, sem_g, sem_s):
    c = lax.axis_index("c")
    s = lax.axis_index("s")
    wbase = (c * 16 + s) * EW

    _zero_vmem_block(zbuf, KS)
    base_r = s * RPW
    for half in range(RPW // KS):
        pltpu.sync_copy(zbuf, hd_sh.at[pl.ds(base_r + half * KS, KS)])
    plsc.subcore_barrier()

    def chunk(ch, _):
        ebase = wbase + ch * SUP
        cps = []
        for j in range(NSUB):
            sl = pl.ds(ebase + j * KS, KS)
            cps.append(pltpu.async_copy(rows_h.at[sl], ridx.at[j], sem_in))
            cps.append(pltpu.async_copy(cols_h.at[sl], cidx.at[j], sem_in))
            cps.append(pltpu.async_copy(vals_h.at[sl], vbuf.at[j], sem_in))
        for cp in cps:
            cp.wait()
        gps = [pltpu.async_copy(gu_h.at[ridx.at[j]], buf_g.at[j], sem_g)
               for j in range(NSUB)]
        for cp in gps:
            cp.wait()
        for j in range(NSUB):
            _scale_rows(buf_g, vbuf, j)
        sps = [pltpu.async_copy(buf_g.at[j], hd_sh.at[cidx.at[j]],
                                sem_s, add=True)
               for j in range(NSUB)]
        for cp in sps:
            cp.wait()
        return 0
    lax.fori_loop(0, NCH, chunk, 0)

    w = c * 16 + s
    fsl = pl.ds(w * (N // NW), N // NW)
    pltpu.sync_copy(fp_h.at[fsl], iidx.at[0])

    def clampb(g, _):
        sl = pl.ds(g * 16, 16)
        iidx[0, sl] = jnp.minimum(iidx[0, sl], NNZ - 1)
        return 0
    lax.fori_loop(0, (N // NW) // 16, clampb, 0)
    pltpu.async_copy(cols_h.at[iidx.at[0]], cbuf.at[0], sem_g).wait()
    pltpu.sync_copy(cbuf.at[0], c1_h.at[fsl])
    pltpu.async_copy(vals_h.at[iidx.at[0]], vvbuf.at[0], sem_g).wait()
    pltpu.sync_copy(vvbuf.at[0], v1_h.at[fsl])

    plsc.subcore_barrier()
    sl_out = pl.ds(s * RPW, RPW)
    pltpu.sync_copy(hd_sh.at[sl_out], hd_h.at[c, sl_out])


def _run_sc_hd(rows, cols, vals, gu, fpm):
    mesh = plsc.VectorSubcoreMesh(core_axis_name="c", subcore_axis_name="s")
    f = pl.kernel(
        _sc_hd_body,
        compiler_params=pltpu.CompilerParams(needs_layout_passes=False),
        out_type=[
            jax.ShapeDtypeStruct((2, N, R), F32),
            jax.ShapeDtypeStruct((N,), jnp.int32),
            jax.ShapeDtypeStruct((N,), F32),
        ],
        mesh=mesh,
        scratch_types=[
            pltpu.VMEM((NSUB, KS), jnp.int32),
            pltpu.VMEM((NSUB, KS), jnp.int32),
            pltpu.VMEM((NSUB, KS), F32),
            pltpu.VMEM((NSUB, KS, R), F32),
            pltpu.VMEM((1, N // NW), jnp.int32),
            pltpu.VMEM((1, N // NW), jnp.int32),
            pltpu.VMEM((1, N // NW), F32),
            pltpu.VMEM((KS, R), F32),
            pltpu.VMEM_SHARED((N, R), F32),
            pltpu.SemaphoreType.DMA,
            pltpu.SemaphoreType.DMA,
            pltpu.SemaphoreType.DMA,
        ],
    )
    return f(rows, cols, vals, gu, fpm)


def _phase_b(igp_ref, ugp_ref, usv_ref, isv_ref, lam_ref, fpp_ref,
             gu_ref, gis_ref, fp_ref):
    inv_coeff = 1.0 / 4096.0
    ig = (igp_ref[0] + igp_ref[1]) * inv_coeff
    ug = (ugp_ref[0] + ugp_ref[1]) * inv_coeff
    gu_ref[...] = (ug + usv_ref[...]) * 0.5
    gis_ref[...] = ((ig + isv_ref[...]) * 0.5) * (1.0 / lam_ref[...])
    fp_ref[...] = jnp.min(fpp_ref[...], axis=0, keepdims=True)


def _run_b(ig_parts, ug_parts, user_sv, item_sv, lambda_mat, fp_parts):
    return pl.pallas_call(
        _phase_b,
        out_shape=[
            jax.ShapeDtypeStruct((N, R), F32),
            jax.ShapeDtypeStruct((N, R), F32),
            jax.ShapeDtypeStruct((1, N), jnp.int32),
        ],
    )(ig_parts, ug_parts, user_sv, item_sv, lambda_mat.reshape(1, R),
      fp_parts)


def _phase_c(na_ref, uf_ref, gis_ref, usvf_ref, hd0_ref, hd1_ref,
             cmin_ref, csum_ref, c1_ref, v1_ref, fp_ref, out_ref):
    w = jnp.dot(na_ref[...], gis_ref[...], preferred_element_type=F32)
    hd = hd0_ref[...] + hd1_ref[...]
    rating = lax.dot_general(w, hd, (((1,), (1,)), ((), ())),
                             preferred_element_type=F32)
    e = lax.dot_general(uf_ref[...], usvf_ref[...], (((1,), (1,)), ((), ())),
                        preferred_element_type=F32)
    cmin = cmin_ref[...]
    inv = 1.0 / (csum_ref[...] - float(N) * cmin + float(N) * 1e-8)
    ne = (e - cmin + 1e-8) * inv
    c1 = c1_ref[...]
    c2 = lax.rem(c1 + 1, N)
    v1 = v1_ref[...]
    has = fp_ref[...] < jnp.int32(NNZ)
    d1 = jnp.where(v1 == 0.0, 1e-5, v1)
    d2 = jnp.where(v1 == 0.0, 1e-5, -v1)
    s1 = jnp.where(has, 0.2 / d1, 0.0)
    s2 = jnp.where(has, 0.2 / d2, 0.0)
    iota = lax.broadcasted_iota(jnp.int32, (BLK, N), 1)
    adj = ne * (jnp.where(iota == c1, s1, 0.0) + jnp.where(iota == c2, s2, 0.0))
    out_ref[...] = rating + adj


def _run_c(norm_adj, uf, gis, user_sv_f, hd0, hd1, cmin, csum, c1, v1, fp):
    return pl.pallas_call(
        _phase_c,
        grid=(GRID,),
        in_specs=[
            pl.BlockSpec((BLK, N), lambda i: (i, 0)),
            pl.BlockSpec((BLK, R), lambda i: (i, 0)),
            pl.BlockSpec((N, R), lambda i: (0, 0)),
            pl.BlockSpec((N, R), lambda i: (0, 0)),
            pl.BlockSpec((N, R), lambda i: (0, 0)),
            pl.BlockSpec((N, R), lambda i: (0, 0)),
            pl.BlockSpec((1, N), lambda i: (0, 0)),
            pl.BlockSpec((1, N), lambda i: (0, 0)),
            pl.BlockSpec((BLK, 1), lambda i: (i, 0)),
            pl.BlockSpec((BLK, 1), lambda i: (i, 0)),
            pl.BlockSpec((BLK, 1), lambda i: (i, 0)),
        ],
        out_specs=pl.BlockSpec((BLK, N), lambda i: (i, 0)),
        out_shape=jax.ShapeDtypeStruct((N, N), F32),
    )(norm_adj, uf, gis, user_sv_f, hd0, hd1, cmin, csum,
      c1.reshape(N, 1), v1.reshape(N, 1), fp.reshape(N, 1))


def kernel(adj_mat, norm_adj, user_sv, item_sv, user_sv_f, item_sv_f,
           pos_values, lambda_mat, lambda_mat_pos, pos_indices):
    rows = pos_indices[0]
    cols = pos_indices[1]

    uf, cmin, csum = _run_a(adj_mat, item_sv_f, user_sv_f, lambda_mat_pos)

    ig_p, ug_p, fp_p = _run_sc_edges(rows, cols, pos_values, item_sv, user_sv)
    gu, gis, fpm = _run_b(ig_p, ug_p, user_sv, item_sv, lambda_mat, fp_p)
    fpm = fpm.reshape(N)
    hd_p, c1, v1 = _run_sc_hd(rows, cols, pos_values, gu, fpm)

    out = _run_c(norm_adj, uf, gis, user_sv_f, hd_p[0], hd_p[1],
                 cmin, csum, c1, v1, fpm)
    return out


# speedup vs baseline: 6.8985x; 1.0012x over previous
import functools
import jax
import jax.numpy as jnp
from jax import lax
from jax.experimental import pallas as pl
from jax.experimental.pallas import tpu as pltpu
from jax.experimental.pallas import tpu_sc as plsc

N = 4096
R = 128
NNZ = 131072
BLK = 256
GRID = N // BLK
F32 = jnp.float32
BIGI = 2 ** 30

NW = 32
EW = NNZ // NW
KS = 128
NSUB = 2
SUP = KS * NSUB
NCH = EW // SUP
RPW = N // 16


def _phase_a(adj_ref, isf_ref, usvf_ref, lp_ref, uf_ref, cmin_ref, csum_ref):
    i = pl.program_id(0)
    uf = jnp.dot(adj_ref[...], isf_ref[...], preferred_element_type=F32)
    uf = uf * (1.0 / lp_ref[...])
    uf_ref[...] = uf
    e = lax.dot_general(uf, usvf_ref[...], (((1,), (1,)), ((), ())),
                        preferred_element_type=F32)
    bmin = jnp.min(e, axis=0, keepdims=True)
    bsum = jnp.sum(e, axis=0, keepdims=True)

    @pl.when(i == 0)
    def _():
        cmin_ref[...] = bmin
        csum_ref[...] = bsum

    @pl.when(i != 0)
    def _():
        cmin_ref[...] = jnp.minimum(cmin_ref[...], bmin)
        csum_ref[...] = csum_ref[...] + bsum


def _run_a(adj_mat, item_sv_f, user_sv_f, lambda_mat_pos):
    return pl.pallas_call(
        _phase_a,
        grid=(GRID,),
        in_specs=[
            pl.BlockSpec((BLK, N), lambda i: (i, 0)),
            pl.BlockSpec((N, R), lambda i: (0, 0)),
            pl.BlockSpec((N, R), lambda i: (0, 0)),
            pl.BlockSpec((1, R), lambda i: (0, 0)),
        ],
        out_specs=[
            pl.BlockSpec((BLK, R), lambda i: (i, 0)),
            pl.BlockSpec((1, N), lambda i: (0, 0)),
            pl.BlockSpec((1, N), lambda i: (0, 0)),
        ],
        out_shape=[
            jax.ShapeDtypeStruct((N, R), F32),
            jax.ShapeDtypeStruct((1, N), F32),
            jax.ShapeDtypeStruct((1, N), F32),
        ],
    )(adj_mat, item_sv_f, user_sv_f, lambda_mat_pos.reshape(1, R))


def _zero_vmem_block(zbuf, nrow):
    def zrow(i, _):
        for j in range(R // 16):
            zbuf[i, pl.ds(j * 16, 16)] = jnp.zeros((16,), F32)
        return 0
    lax.fori_loop(0, nrow, zrow, 0)


def _scale_rows(buf, vbuf, j):
    def body(g, _):
        v16 = vbuf[j, pl.ds(g * 16, 16)]
        for l in range(16):
            v = v16[l]
            rv = buf.at[j, g * 16 + l]
            for t in range(R // 16):
                sl = pl.ds(t * 16, 16)
                rv[sl] = rv[sl] * v
        return 0
    lax.fori_loop(0, KS // 16, body, 0)


def _fp_update(fp_loc, ridx, j, ebase):
    def body(g, _):
        r16 = ridx[j, pl.ds(g * 16, 16)]
        e16 = (ebase + g * 16) + lax.broadcasted_iota(jnp.int32, (16,), 0)
        idx = lax.broadcasted_iota(jnp.int32, (16,), 0)
        for l in range(16):
            m = idx == l
            cur = plsc.load_gather(fp_loc, [r16], mask=m)
            plsc.store_scatter(fp_loc, [r16], jnp.minimum(cur, e16), mask=m)
        return 0
    lax.fori_loop(0, KS // 16, body, 0)


def _sc_edges_body(rows_h, cols_h, vals_h, item_h, user_h,
                   ig_h, ug_h, fp_h,
                   ridx, cidx, vbuf, buf, fp_loc, zbuf,
                   acc_sh, sem_in, sem_g, sem_s):
    c = lax.axis_index("c")
    s = lax.axis_index("s")
    w = c * 16 + s
    wbase = w * EW
    base_r = s * RPW
    sl_out = pl.ds(base_r, RPW)

    _zero_vmem_block(zbuf, KS)

    def fz(i, _):
        fp_loc[pl.ds(i * 16, 16)] = jnp.full((16,), BIGI, jnp.int32)
        return 0
    lax.fori_loop(0, N // 16, fz, 0)

    for phase in range(2):
        for half in range(RPW // KS):
            pltpu.sync_copy(zbuf, acc_sh.at[pl.ds(base_r + half * KS, KS)])
        plsc.subcore_barrier()

        def chunk(ch, _):
            ebase = wbase + ch * SUP
            cps = []
            for j in range(NSUB):
                sl = pl.ds(ebase + j * KS, KS)
                cps.append(pltpu.async_copy(rows_h.at[sl], ridx.at[j], sem_in))
                cps.append(pltpu.async_copy(cols_h.at[sl], cidx.at[j], sem_in))
                cps.append(pltpu.async_copy(vals_h.at[sl], vbuf.at[j], sem_in))
            for cp in cps:
                cp.wait()
            tbl = item_h if phase == 0 else user_h
            gi = cidx if phase == 0 else ridx
            si = ridx if phase == 0 else cidx
            gps = [pltpu.async_copy(tbl.at[gi.at[j]], buf.at[j], sem_g)
                   for j in range(NSUB)]
            for cp in gps:
                cp.wait()
            for j in range(NSUB):
                _scale_rows(buf, vbuf, j)
                if phase == 0:
                    _fp_update(fp_loc, ridx, j, ebase + j * KS)
            sps = [pltpu.async_copy(buf.at[j], acc_sh.at[si.at[j]],
                                    sem_s, add=True)
                   for j in range(NSUB)]
            for cp in sps:
                cp.wait()
            return 0
        lax.fori_loop(0, NCH, chunk, 0)
        plsc.subcore_barrier()
        dst = ug_h if phase == 0 else ig_h
        pltpu.sync_copy(acc_sh.at[sl_out], dst.at[c, sl_out])
        plsc.subcore_barrier()

    pltpu.sync_copy(fp_loc, fp_h.at[w])


def _run_sc_edges(rows, cols, vals, item_sv, user_sv):
    mesh = plsc.VectorSubcoreMesh(core_axis_name="c", subcore_axis_name="s")
    f = pl.kernel(
        _sc_edges_body,
        compiler_params=pltpu.CompilerParams(needs_layout_passes=False),
        out_type=[
            jax.ShapeDtypeStruct((2, N, R), F32),
            jax.ShapeDtypeStruct((2, N, R), F32),
            jax.ShapeDtypeStruct((NW, N), jnp.int32),
        ],
        mesh=mesh,
        scratch_types=[
            pltpu.VMEM((NSUB, KS), jnp.int32),
            pltpu.VMEM((NSUB, KS), jnp.int32),
            pltpu.VMEM((NSUB, KS), F32),
            pltpu.VMEM((NSUB, KS, R), F32),
            pltpu.VMEM((N,), jnp.int32),
            pltpu.VMEM((KS, R), F32),
            pltpu.VMEM_SHARED((N, R), F32),
            pltpu.SemaphoreType.DMA,
            pltpu.SemaphoreType.DMA,
            pltpu.SemaphoreType.DMA,
        ],
    )
    return f(rows, cols, vals, item_sv, user_sv)


def _sc_hd_body(rows_h, cols_h, vals_h, gu_h, fp_h,
                hd_h, c1_h, v1_h,
                ridx, cidx, vbuf, buf_g, iidx, cbuf, vvbuf, zbuf,
                hd_sh, sem_in, sem_g, sem_s):
    c = lax.axis_index("c")
    s = lax.axis_index("s")
    wbase = (c * 16 + s) * EW

    _zero_vmem_block(zbuf, KS)
    base_r = s * RPW
    for half in range(RPW // KS):
        pltpu.sync_copy(zbuf, hd_sh.at[pl.ds(base_r + half * KS, KS)])
    plsc.subcore_barrier()

    def chunk(ch, _):
        ebase = wbase + ch * SUP
        cps = []
        for j in range(NSUB):
            sl = pl.ds(ebase + j * KS, KS)
            cps.append(pltpu.async_copy(rows_h.at[sl], ridx.at[j], sem_in))
            cps.append(pltpu.async_copy(cols_h.at[sl], cidx.at[j], sem_in))
            cps.append(pltpu.async_copy(vals_h.at[sl], vbuf.at[j], sem_in))
        for cp in cps:
            cp.wait()
        gps = [pltpu.async_copy(gu_h.at[ridx.at[j]], buf_g.at[j], sem_g)
               for j in range(NSUB)]
        for cp in gps:
            cp.wait()
        for j in range(NSUB):
            _scale_rows(buf_g, vbuf, j)
        sps = [pltpu.async_copy(buf_g.at[j], hd_sh.at[cidx.at[j]],
                                sem_s, add=True)
               for j in range(NSUB)]
        for cp in sps:
            cp.wait()
        return 0
    lax.fori_loop(0, NCH, chunk, 0)

    w = c * 16 + s
    fsl = pl.ds(w * (N // NW), N // NW)
    pltpu.sync_copy(fp_h.at[fsl], iidx.at[0])

    def clampb(g, _):
        sl = pl.ds(g * 16, 16)
        iidx[0, sl] = jnp.minimum(iidx[0, sl], NNZ - 1)
        return 0
    lax.fori_loop(0, (N // NW) // 16, clampb, 0)
    pltpu.async_copy(cols_h.at[iidx.at[0]], cbuf.at[0], sem_g).wait()
    pltpu.sync_copy(cbuf.at[0], c1_h.at[fsl])
    pltpu.async_copy(vals_h.at[iidx.at[0]], vvbuf.at[0], sem_g).wait()
    pltpu.sync_copy(vvbuf.at[0], v1_h.at[fsl])

    plsc.subcore_barrier()
    sl_out = pl.ds(s * RPW, RPW)
    pltpu.sync_copy(hd_sh.at[sl_out], hd_h.at[c, sl_out])


def _run_sc_hd(rows, cols, vals, gu, fpm):
    mesh = plsc.VectorSubcoreMesh(core_axis_name="c", subcore_axis_name="s")
    f = pl.kernel(
        _sc_hd_body,
        compiler_params=pltpu.CompilerParams(needs_layout_passes=False),
        out_type=[
            jax.ShapeDtypeStruct((2, N, R), F32),
            jax.ShapeDtypeStruct((N,), jnp.int32),
            jax.ShapeDtypeStruct((N,), F32),
        ],
        mesh=mesh,
        scratch_types=[
            pltpu.VMEM((NSUB, KS), jnp.int32),
            pltpu.VMEM((NSUB, KS), jnp.int32),
            pltpu.VMEM((NSUB, KS), F32),
            pltpu.VMEM((NSUB, KS, R), F32),
            pltpu.VMEM((1, N // NW), jnp.int32),
            pltpu.VMEM((1, N // NW), jnp.int32),
            pltpu.VMEM((1, N // NW), F32),
            pltpu.VMEM((KS, R), F32),
            pltpu.VMEM_SHARED((N, R), F32),
            pltpu.SemaphoreType.DMA,
            pltpu.SemaphoreType.DMA,
            pltpu.SemaphoreType.DMA,
        ],
    )
    return f(rows, cols, vals, gu, fpm)


def _phase_b(igp_ref, ugp_ref, usv_ref, isv_ref, lam_ref, fpp_ref,
             gu_ref, gis_ref, fp_ref):
    inv_coeff = 1.0 / 4096.0
    ig = (igp_ref[0] + igp_ref[1]) * inv_coeff
    ug = (ugp_ref[0] + ugp_ref[1]) * inv_coeff
    gu_ref[...] = (ug + usv_ref[...]) * 0.5
    gis_ref[...] = ((ig + isv_ref[...]) * 0.5) * (1.0 / lam_ref[...])
    fp_ref[...] = jnp.min(fpp_ref[...], axis=0, keepdims=True)


def _run_b(ig_parts, ug_parts, user_sv, item_sv, lambda_mat, fp_parts):
    return pl.pallas_call(
        _phase_b,
        out_shape=[
            jax.ShapeDtypeStruct((N, R), F32),
            jax.ShapeDtypeStruct((N, R), F32),
            jax.ShapeDtypeStruct((1, N), jnp.int32),
        ],
    )(ig_parts, ug_parts, user_sv, item_sv, lambda_mat.reshape(1, R),
      fp_parts)


def _phase_c(na_ref, uf_ref, gis_ref, usvf_ref, hd0_ref, hd1_ref,
             cmin_ref, csum_ref, c1_ref, v1_ref, fp_ref, out_ref):
    w = jnp.dot(na_ref[...], gis_ref[...], preferred_element_type=F32)
    hd = hd0_ref[...] + hd1_ref[...]
    rating = lax.dot_general(w, hd, (((1,), (1,)), ((), ())),
                             preferred_element_type=F32)
    e = lax.dot_general(uf_ref[...], usvf_ref[...], (((1,), (1,)), ((), ())),
                        preferred_element_type=F32)
    cmin = cmin_ref[...]
    inv = 1.0 / (csum_ref[...] - float(N) * cmin + float(N) * 1e-8)
    ne = (e - cmin + 1e-8) * inv
    c1 = c1_ref[...]
    c2 = lax.rem(c1 + 1, N)
    v1 = v1_ref[...]
    has = fp_ref[...] < jnp.int32(NNZ)
    d1 = jnp.where(v1 == 0.0, 1e-5, v1)
    d2 = jnp.where(v1 == 0.0, 1e-5, -v1)
    s1 = jnp.where(has, 0.2 / d1, 0.0)
    s2 = jnp.where(has, 0.2 / d2, 0.0)
    iota = lax.broadcasted_iota(jnp.int32, (BLK, N), 1)
    adj = ne * (jnp.where(iota == c1, s1, 0.0) + jnp.where(iota == c2, s2, 0.0))
    out_ref[...] = rating + adj


def _run_c(norm_adj, uf, gis, user_sv_f, hd0, hd1, cmin, csum, c1, v1, fp):
    return pl.pallas_call(
        _phase_c,
        grid=(GRID,),
        in_specs=[
            pl.BlockSpec((BLK, N), lambda i: (i, 0)),
            pl.BlockSpec((BLK, R), lambda i: (i, 0)),
            pl.BlockSpec((N, R), lambda i: (0, 0)),
            pl.BlockSpec((N, R), lambda i: (0, 0)),
            pl.BlockSpec((N, R), lambda i: (0, 0)),
            pl.BlockSpec((N, R), lambda i: (0, 0)),
            pl.BlockSpec((1, N), lambda i: (0, 0)),
            pl.BlockSpec((1, N), lambda i: (0, 0)),
            pl.BlockSpec((BLK, 1), lambda i: (i, 0)),
            pl.BlockSpec((BLK, 1), lambda i: (i, 0)),
            pl.BlockSpec((BLK, 1), lambda i: (i, 0)),
        ],
        out_specs=pl.BlockSpec((BLK, N), lambda i: (i, 0)),
        out_shape=jax.ShapeDtypeStruct((N, N), F32),
    )(norm_adj, uf, gis, user_sv_f, hd0, hd1, cmin, csum,
      c1.reshape(N, 1), v1.reshape(N, 1), fp.reshape(N, 1))


def kernel(adj_mat, norm_adj, user_sv, item_sv, user_sv_f, item_sv_f,
           pos_values, lambda_mat, lambda_mat_pos, pos_indices):
    rows = pos_indices[0]
    cols = pos_indices[1]

    uf, cmin, csum = _run_a(adj_mat, item_sv_f, user_sv_f, lambda_mat_pos)

    ig_p, ug_p, fp_p = _run_sc_edges(rows, cols, pos_values, item_sv, user_sv)
    gu, gis, fpm = _run_b(ig_p, ug_p, user_sv, item_sv, lambda_mat, fp_p)
    fpm = fpm.reshape(N)
    hd_p, c1, v1 = _run_sc_hd(rows, cols, pos_values, gu, fpm)

    out = _run_c(norm_adj, uf, gis, user_sv_f, hd_p[0], hd_p[1],
                 cmin, csum, c1, v1, fpm)
    return out
